# baseline (speedup 1.0000x reference)
import functools

import jax
import jax.numpy as jnp
from jax import lax
from jax.experimental import pallas as pl
from jax.experimental.pallas import tpu as pltpu
from jax.experimental.pallas import tpu_sc as plsc

N_NODES = 100000
N_EDGES = 3200000

N_PAD = 102400
E_PAD = 3276800
LANES = 128
ROWS_TOTAL = E_PAD // LANES
NUM_TILES = 32
ROWS_PER_TILE = ROWS_TOTAL // NUM_TILES
NSLICE = N_PAD // 16

RB = 16
NBLK = ROWS_PER_TILE // RB
NQUAD = NBLK // 4
RBD = 32
NBLKD = ROWS_PER_TILE // RBD

_mesh = plsc.VectorSubcoreMesh(core_axis_name="c", subcore_axis_name="s")


def _deg_body(dst_hbm, zeros_hbm, out_hbm, idxb, acc, sem_i):
    cid = lax.axis_index("c")
    sid = lax.axis_index("s")
    wid = cid * 16 + sid
    pltpu.sync_copy(zeros_hbm, acc)
    rbase = wid * ROWS_PER_TILE
    ones = jnp.ones((16,), jnp.float32)
    pltpu.async_copy(dst_hbm.at[pl.ds(rbase, RBD)], idxb.at[0], sem_i)

    def _count(slot):
        for r in range(RBD):
            for j in range(LANES // 16):
                d16 = idxb[slot, r, pl.ds(j * 16, 16)]
                plsc.addupdate_scatter(acc, [d16], ones)

    def pair(ii, carry):
        b0 = ii * 2
        row0 = rbase + b0 * RBD
        pltpu.make_async_copy(dst_hbm.at[pl.ds(row0, RBD)], idxb.at[0],
                              sem_i).wait()
        pltpu.async_copy(dst_hbm.at[pl.ds(row0 + RBD, RBD)], idxb.at[1],
                         sem_i)
        _count(0)
        pltpu.make_async_copy(dst_hbm.at[pl.ds(row0 + RBD, RBD)], idxb.at[1],
                              sem_i).wait()
        pltpu.async_copy(dst_hbm.at[pl.ds(row0 + 2 * RBD, RBD)], idxb.at[0],
                         sem_i)
        _count(1)
        return carry

    lax.fori_loop(0, (NBLKD - 1) // 2, pair, 0)
    pltpu.make_async_copy(dst_hbm.at[pl.ds(rbase + (NBLKD - 1) * RBD, RBD)],
                          idxb.at[0], sem_i).wait()
    _count(0)
    pltpu.sync_copy(acc, out_hbm.at[wid])


_deg_kernel = functools.partial(
    pl.kernel,
    out_type=jax.ShapeDtypeStruct((NUM_TILES, N_PAD), jnp.float32),
    mesh=_mesh,
    compiler_params=pltpu.CompilerParams(needs_layout_passes=False),
    scratch_types=[
        pltpu.VMEM((2, RBD, LANES), jnp.int32),
        pltpu.VMEM((N_PAD,), jnp.float32),
        pltpu.SemaphoreType.DMA,
    ],
)(_deg_body)


def _gs_body(src_hbm, dst_hbm, g_hbm, zeros_hbm, out_hbm,
             sbuf, dbuf, vbuf, gcopy, tsh, sem_i, sem_s):
    cid = lax.axis_index("c")
    sid = lax.axis_index("s")
    wid = cid * 16 + sid
    nbase = sid * NSLICE
    rbase = wid * ROWS_PER_TILE

    def _fetch(b, slot):
        row0 = rbase + b * RB
        pltpu.async_copy(src_hbm.at[pl.ds(row0, RB)], sbuf.at[slot], sem_i)
        pltpu.async_copy(dst_hbm.at[pl.ds(row0, RB)], dbuf.at[slot], sem_i)

    def _wait_fetch(b, slot):
        row0 = rbase + b * RB
        pltpu.make_async_copy(src_hbm.at[pl.ds(row0, RB)], sbuf.at[slot],
                              sem_i).wait()
        pltpu.make_async_copy(dst_hbm.at[pl.ds(row0, RB)], dbuf.at[slot],
                              sem_i).wait()

    def _gather(slot, q):
        for r in range(RB):
            for j in range(LANES // 16):
                s16 = sbuf[slot, r, pl.ds(j * 16, 16)]
                vbuf[q, r, pl.ds(j * 16, 16)] = plsc.load_gather(gcopy, [s16])

    def _issue_scatter(slot, q):
        for r in range(RB):
            pltpu.async_copy(vbuf.at[q, r], tsh.at[dbuf.at[slot, r]], sem_s,
                             add=True)

    def _wait_scatter(slot, q):
        for r in range(RB):
            pltpu.make_async_copy(vbuf.at[q, r], tsh.at[dbuf.at[slot, r]],
                                  sem_s).wait()

    for b in range(3):
        _fetch(b, b)
    pltpu.async_copy(zeros_hbm.at[pl.ds(nbase, NSLICE)],
                     tsh.at[pl.ds(nbase, NSLICE)], sem_s)
    pltpu.async_copy(g_hbm, gcopy, sem_s)
    pltpu.make_async_copy(zeros_hbm.at[pl.ds(nbase, NSLICE)],
                          tsh.at[pl.ds(nbase, NSLICE)], sem_s).wait()
    pltpu.make_async_copy(g_hbm, gcopy, sem_s).wait()
    plsc.subcore_barrier()

    def quad(ii, carry):
        b0 = ii * 4
        for k in range(4):
            slot = k
            q = k % 2
            _wait_fetch(b0 + k, slot)
            if k == 0:
                @pl.when(ii > 0)
                def _():
                    _wait_scatter(3, 1)
            else:
                _wait_scatter(k - 1, (k - 1) % 2)

            @pl.when(b0 + k + 3 < NBLK)
            def _():
                _fetch(b0 + k + 3, (k + 3) % 4)

            _gather(slot, q)
            _issue_scatter(slot, q)
        return carry

    lax.fori_loop(0, NQUAD, quad, 0)
    for b in range(NQUAD * 4, NBLK):
        slot = b % 4
        q = b % 2
        _wait_fetch(b, slot)
        _wait_scatter((b - 1) % 4, (b - 1) % 2)
        _gather(slot, q)
        _issue_scatter(slot, q)
    _wait_scatter((NBLK - 1) % 4, (NBLK - 1) % 2)
    plsc.subcore_barrier()
    pltpu.sync_copy(tsh.at[pl.ds(nbase, NSLICE)],
                    out_hbm.at[cid, pl.ds(nbase, NSLICE)])


_gs_kernel = functools.partial(
    pl.kernel,
    out_type=jax.ShapeDtypeStruct((2, N_PAD), jnp.float32),
    mesh=_mesh,
    compiler_params=pltpu.CompilerParams(needs_layout_passes=False),
    scratch_types=[
        pltpu.VMEM((4, RB, LANES), jnp.int32),
        pltpu.VMEM((4, RB, LANES), jnp.int32),
        pltpu.VMEM((2, RB, LANES), jnp.float32),
        pltpu.VMEM((N_PAD,), jnp.float32),
        pltpu.VMEM_SHARED((N_PAD,), jnp.float32),
        pltpu.SemaphoreType.DMA,
        pltpu.SemaphoreType.DMA,
    ],
)(_gs_body)


_R2 = N_PAD // LANES


def _prep_body(deg32_ref, x_ref, dinv_ref, g_ref):
    deg = jnp.sum(deg32_ref[...], axis=0) + 1.0
    dinv = lax.rsqrt(deg)
    dinv_ref[...] = dinv
    g_ref[...] = dinv * x_ref[...]


_prep_call = pl.pallas_call(
    _prep_body,
    out_shape=(jax.ShapeDtypeStruct((_R2, LANES), jnp.float32),
               jax.ShapeDtypeStruct((_R2, LANES), jnp.float32)),
)


def _mid_body(t1_ref, dinv_ref, g_ref, w1_ref, b1_ref, w2_ref,
              h2_ref, g2_ref):
    dinv = dinv_ref[...]
    s1 = dinv * (t1_ref[0] + t1_ref[1] + g_ref[...])
    acc = jnp.zeros_like(s1)
    for k in range(16):
        acc = acc + jnp.maximum(s1 * w1_ref[0, k] + b1_ref[k], 0.0) * w2_ref[k, 0]
    h2_ref[...] = acc
    g2_ref[...] = dinv * acc


_mid_call = pl.pallas_call(
    _mid_body,
    in_specs=[
        pl.BlockSpec(memory_space=pltpu.VMEM),
        pl.BlockSpec(memory_space=pltpu.VMEM),
        pl.BlockSpec(memory_space=pltpu.VMEM),
        pl.BlockSpec(memory_space=pltpu.SMEM),
        pl.BlockSpec(memory_space=pltpu.SMEM),
        pl.BlockSpec(memory_space=pltpu.SMEM),
    ],
    out_shape=(jax.ShapeDtypeStruct((_R2, LANES), jnp.float32),
               jax.ShapeDtypeStruct((_R2, LANES), jnp.float32)),
)


def _fin_body(t2_ref, dinv_ref, g2_ref, b2_ref, out_ref):
    out_ref[...] = dinv_ref[...] * (t2_ref[0] + t2_ref[1] + g2_ref[...]) + b2_ref[0]


_fin_call = pl.pallas_call(
    _fin_body,
    in_specs=[
        pl.BlockSpec(memory_space=pltpu.VMEM),
        pl.BlockSpec(memory_space=pltpu.VMEM),
        pl.BlockSpec(memory_space=pltpu.VMEM),
        pl.BlockSpec(memory_space=pltpu.SMEM),
    ],
    out_shape=jax.ShapeDtypeStruct((_R2, LANES), jnp.float32),
)


def kernel(x, edge_index, W1, b1, W2, b2):
    ei = edge_index.astype(jnp.int32)
    pad = jnp.full((E_PAD - N_EDGES,), N_NODES, jnp.int32)
    src = jnp.concatenate([ei[0], pad]).reshape(ROWS_TOTAL, LANES)
    dst = jnp.concatenate([ei[1], pad]).reshape(ROWS_TOTAL, LANES)
    xp = jnp.pad(x[:, 0], (0, N_PAD - N_NODES))
    zeros = jnp.zeros((N_PAD,), jnp.float32)

    deg32 = _deg_kernel(dst, zeros)
    dinv, g = _prep_call(deg32.reshape(NUM_TILES, _R2, LANES),
                         xp.reshape(_R2, LANES))
    t1 = _gs_kernel(src, dst, g.reshape(N_PAD), zeros)
    h2, g2 = _mid_call(t1.reshape(2, _R2, LANES), dinv, g, W1, b1, W2)
    t2 = _gs_kernel(src, dst, g2.reshape(N_PAD), zeros)
    out = _fin_call(t2.reshape(2, _R2, LANES), dinv, g2, b2)
    return out.reshape(N_PAD)[:N_NODES].reshape(N_NODES, 1)


# speedup vs baseline: 1.0211x; 1.0211x over previous
import functools

import jax
import jax.numpy as jnp
from jax import lax
from jax.experimental import pallas as pl
from jax.experimental.pallas import tpu as pltpu
from jax.experimental.pallas import tpu_sc as plsc

N_NODES = 100000
N_EDGES = 3200000

N_PAD = 102400
E_PAD = 3276800
LANES = 128
ROWS_TOTAL = E_PAD // LANES
NUM_TILES = 32
ROWS_PER_TILE = ROWS_TOTAL // NUM_TILES
NSLICE = N_PAD // 16

RB = 8
NBLK = ROWS_PER_TILE // RB
NQUAD = NBLK // 4
RBD = 32
NBLKD = ROWS_PER_TILE // RBD

_mesh = plsc.VectorSubcoreMesh(core_axis_name="c", subcore_axis_name="s")


def _deg_body(dst_hbm, zeros_hbm, out_hbm, idxb, acc, sem_i):
    cid = lax.axis_index("c")
    sid = lax.axis_index("s")
    wid = cid * 16 + sid
    pltpu.sync_copy(zeros_hbm, acc)
    rbase = wid * ROWS_PER_TILE
    ones = jnp.ones((16,), jnp.float32)
    pltpu.async_copy(dst_hbm.at[pl.ds(rbase, RBD)], idxb.at[0], sem_i)

    def _count(slot):
        for r in range(RBD):
            for j in range(LANES // 16):
                d16 = idxb[slot, r, pl.ds(j * 16, 16)]
                plsc.addupdate_scatter(acc, [d16], ones)

    def pair(ii, carry):
        b0 = ii * 2
        row0 = rbase + b0 * RBD
        pltpu.make_async_copy(dst_hbm.at[pl.ds(row0, RBD)], idxb.at[0],
                              sem_i).wait()
        pltpu.async_copy(dst_hbm.at[pl.ds(row0 + RBD, RBD)], idxb.at[1],
                         sem_i)
        _count(0)
        pltpu.make_async_copy(dst_hbm.at[pl.ds(row0 + RBD, RBD)], idxb.at[1],
                              sem_i).wait()
        pltpu.async_copy(dst_hbm.at[pl.ds(row0 + 2 * RBD, RBD)], idxb.at[0],
                         sem_i)
        _count(1)
        return carry

    lax.fori_loop(0, (NBLKD - 1) // 2, pair, 0)
    pltpu.make_async_copy(dst_hbm.at[pl.ds(rbase + (NBLKD - 1) * RBD, RBD)],
                          idxb.at[0], sem_i).wait()
    _count(0)
    pltpu.sync_copy(acc, out_hbm.at[wid])


_deg_kernel = functools.partial(
    pl.kernel,
    out_type=jax.ShapeDtypeStruct((NUM_TILES, N_PAD), jnp.float32),
    mesh=_mesh,
    compiler_params=pltpu.CompilerParams(needs_layout_passes=False),
    scratch_types=[
        pltpu.VMEM((2, RBD, LANES), jnp.int32),
        pltpu.VMEM((N_PAD,), jnp.float32),
        pltpu.SemaphoreType.DMA,
    ],
)(_deg_body)


def _gs_body(src_hbm, dst_hbm, g_hbm, zeros_hbm, out_hbm,
             sbuf, dbuf, vbuf, gcopy, tsh, sem_i, sem_s):
    cid = lax.axis_index("c")
    sid = lax.axis_index("s")
    wid = cid * 16 + sid
    nbase = sid * NSLICE
    rbase = wid * ROWS_PER_TILE

    def _fetch(b, slot):
        row0 = rbase + b * RB
        pltpu.async_copy(src_hbm.at[pl.ds(row0, RB)], sbuf.at[slot], sem_i)
        pltpu.async_copy(dst_hbm.at[pl.ds(row0, RB)], dbuf.at[slot], sem_i)

    def _wait_fetch(b, slot):
        row0 = rbase + b * RB
        pltpu.make_async_copy(src_hbm.at[pl.ds(row0, RB)], sbuf.at[slot],
                              sem_i).wait()
        pltpu.make_async_copy(dst_hbm.at[pl.ds(row0, RB)], dbuf.at[slot],
                              sem_i).wait()

    def _gather(slot, q):
        for r in range(RB):
            for j in range(LANES // 16):
                s16 = sbuf[slot, r, pl.ds(j * 16, 16)]
                vbuf[q, r, pl.ds(j * 16, 16)] = plsc.load_gather(gcopy, [s16])

    def _issue_scatter(slot, q):
        for r in range(RB):
            pltpu.async_copy(vbuf.at[q, r], tsh.at[dbuf.at[slot, r]], sem_s,
                             add=True)

    def _wait_scatter(slot, q):
        for r in range(RB):
            pltpu.make_async_copy(vbuf.at[q, r], tsh.at[dbuf.at[slot, r]],
                                  sem_s).wait()

    for b in range(3):
        _fetch(b, b)
    pltpu.async_copy(zeros_hbm.at[pl.ds(nbase, NSLICE)],
                     tsh.at[pl.ds(nbase, NSLICE)], sem_s)
    pltpu.async_copy(g_hbm, gcopy, sem_s)
    pltpu.make_async_copy(zeros_hbm.at[pl.ds(nbase, NSLICE)],
                          tsh.at[pl.ds(nbase, NSLICE)], sem_s).wait()
    pltpu.make_async_copy(g_hbm, gcopy, sem_s).wait()
    plsc.subcore_barrier()

    def quad(ii, carry):
        b0 = ii * 4
        for k in range(4):
            slot = k
            q = k % 2
            _wait_fetch(b0 + k, slot)
            if k == 0:
                @pl.when(ii > 0)
                def _():
                    _wait_scatter(3, 1)
            else:
                _wait_scatter(k - 1, (k - 1) % 2)

            @pl.when(b0 + k + 3 < NBLK)
            def _():
                _fetch(b0 + k + 3, (k + 3) % 4)

            _gather(slot, q)
            _issue_scatter(slot, q)
        return carry

    lax.fori_loop(0, NQUAD, quad, 0)
    for b in range(NQUAD * 4, NBLK):
        slot = b % 4
        q = b % 2
        _wait_fetch(b, slot)
        _wait_scatter((b - 1) % 4, (b - 1) % 2)
        _gather(slot, q)
        _issue_scatter(slot, q)
    _wait_scatter((NBLK - 1) % 4, (NBLK - 1) % 2)
    plsc.subcore_barrier()
    pltpu.sync_copy(tsh.at[pl.ds(nbase, NSLICE)],
                    out_hbm.at[cid, pl.ds(nbase, NSLICE)])


_gs_kernel = functools.partial(
    pl.kernel,
    out_type=jax.ShapeDtypeStruct((2, N_PAD), jnp.float32),
    mesh=_mesh,
    compiler_params=pltpu.CompilerParams(needs_layout_passes=False),
    scratch_types=[
        pltpu.VMEM((4, RB, LANES), jnp.int32),
        pltpu.VMEM((4, RB, LANES), jnp.int32),
        pltpu.VMEM((2, RB, LANES), jnp.float32),
        pltpu.VMEM((N_PAD,), jnp.float32),
        pltpu.VMEM_SHARED((N_PAD,), jnp.float32),
        pltpu.SemaphoreType.DMA,
        pltpu.SemaphoreType.DMA,
    ],
)(_gs_body)


_R2 = N_PAD // LANES


def _prep_body(deg32_ref, x_ref, dinv_ref, g_ref):
    deg = jnp.sum(deg32_ref[...], axis=0) + 1.0
    dinv = lax.rsqrt(deg)
    dinv_ref[...] = dinv
    g_ref[...] = dinv * x_ref[...]


_prep_call = pl.pallas_call(
    _prep_body,
    out_shape=(jax.ShapeDtypeStruct((_R2, LANES), jnp.float32),
               jax.ShapeDtypeStruct((_R2, LANES), jnp.float32)),
)


def _mid_body(t1_ref, dinv_ref, g_ref, w1_ref, b1_ref, w2_ref,
              h2_ref, g2_ref):
    dinv = dinv_ref[...]
    s1 = dinv * (t1_ref[0] + t1_ref[1] + g_ref[...])
    acc = jnp.zeros_like(s1)
    for k in range(16):
        acc = acc + jnp.maximum(s1 * w1_ref[0, k] + b1_ref[k], 0.0) * w2_ref[k, 0]
    h2_ref[...] = acc
    g2_ref[...] = dinv * acc


_mid_call = pl.pallas_call(
    _mid_body,
    in_specs=[
        pl.BlockSpec(memory_space=pltpu.VMEM),
        pl.BlockSpec(memory_space=pltpu.VMEM),
        pl.BlockSpec(memory_space=pltpu.VMEM),
        pl.BlockSpec(memory_space=pltpu.SMEM),
        pl.BlockSpec(memory_space=pltpu.SMEM),
        pl.BlockSpec(memory_space=pltpu.SMEM),
    ],
    out_shape=(jax.ShapeDtypeStruct((_R2, LANES), jnp.float32),
               jax.ShapeDtypeStruct((_R2, LANES), jnp.float32)),
)


def _fin_body(t2_ref, dinv_ref, g2_ref, b2_ref, out_ref):
    out_ref[...] = dinv_ref[...] * (t2_ref[0] + t2_ref[1] + g2_ref[...]) + b2_ref[0]


_fin_call = pl.pallas_call(
    _fin_body,
    in_specs=[
        pl.BlockSpec(memory_space=pltpu.VMEM),
        pl.BlockSpec(memory_space=pltpu.VMEM),
        pl.BlockSpec(memory_space=pltpu.VMEM),
        pl.BlockSpec(memory_space=pltpu.SMEM),
    ],
    out_shape=jax.ShapeDtypeStruct((_R2, LANES), jnp.float32),
)


def kernel(x, edge_index, W1, b1, W2, b2):
    ei = edge_index.astype(jnp.int32)
    pad = jnp.full((E_PAD - N_EDGES,), N_NODES, jnp.int32)
    src = jnp.concatenate([ei[0], pad]).reshape(ROWS_TOTAL, LANES)
    dst = jnp.concatenate([ei[1], pad]).reshape(ROWS_TOTAL, LANES)
    xp = jnp.pad(x[:, 0], (0, N_PAD - N_NODES))
    zeros = jnp.zeros((N_PAD,), jnp.float32)

    deg32 = _deg_kernel(dst, zeros)
    dinv, g = _prep_call(deg32.reshape(NUM_TILES, _R2, LANES),
                         xp.reshape(_R2, LANES))
    t1 = _gs_kernel(src, dst, g.reshape(N_PAD), zeros)
    h2, g2 = _mid_call(t1.reshape(2, _R2, LANES), dinv, g, W1, b1, W2)
    t2 = _gs_kernel(src, dst, g2.reshape(N_PAD), zeros)
    out = _fin_call(t2.reshape(2, _R2, LANES), dinv, g2, b2)
    return out.reshape(N_PAD)[:N_NODES].reshape(N_NODES, 1)


# speedup vs baseline: 1.6947x; 1.6597x over previous
import functools

import jax
import jax.numpy as jnp
from jax import lax
from jax.experimental import pallas as pl
from jax.experimental.pallas import tpu as pltpu
from jax.experimental.pallas import tpu_sc as plsc

N_NODES = 100000
N_EDGES = 3200000

N_PAD = 102400
LANES = 128
ROWS_TOTAL = N_EDGES // LANES
NUM_TILES = 32
ROWS_PER_TILE = 800
NSLICE = N_PAD // 16

RB = 8
NBLK = ROWS_PER_TILE // RB
NBLK_LAST = 200 // RB
NQUAD = NBLK // 4

_mesh = plsc.VectorSubcoreMesh(core_axis_name="c", subcore_axis_name="s")


def _deg_body(dst_hbm, zeros_hbm, out_hbm, idxb, acc, sem_i):
    cid = lax.axis_index("c")
    sid = lax.axis_index("s")
    wid = cid * 16 + sid
    is_last = wid == NUM_TILES - 1
    nblk_w = jnp.where(is_last, NBLK_LAST, NBLK)
    rbase = wid * ROWS_PER_TILE
    ones = jnp.ones((16,), jnp.float32)

    def _fetch(b, slot):
        pltpu.async_copy(dst_hbm.at[pl.ds(rbase + b * RB, RB)], idxb.at[slot],
                         sem_i)

    def _wait_fetch(b, slot):
        pltpu.make_async_copy(dst_hbm.at[pl.ds(rbase + b * RB, RB)],
                              idxb.at[slot], sem_i).wait()

    def _count(slot):
        for r in range(RB):
            for j in range(LANES // 16):
                d16 = idxb[slot, r, pl.ds(j * 16, 16)]
                plsc.addupdate_scatter(acc, [d16], ones)

    for b in range(3):
        _fetch(b, b)
    pltpu.sync_copy(zeros_hbm, acc)

    def quad(ii, carry):
        b0 = ii * 4
        for k in range(4):
            _wait_fetch(b0 + k, k)

            @pl.when(b0 + k + 3 < nblk_w)
            def _():
                _fetch(b0 + k + 3, (k + 3) % 4)

            _count(k)
        return carry

    nq_w = jnp.where(is_last, NBLK_LAST // 4, NQUAD)
    lax.fori_loop(0, nq_w, quad, 0)

    @pl.when(is_last)
    def _():
        _wait_fetch(NBLK_LAST - 1, (NBLK_LAST - 1) % 4)
        _count((NBLK_LAST - 1) % 4)

    pltpu.sync_copy(acc, out_hbm.at[wid])


_deg_kernel = functools.partial(
    pl.kernel,
    out_type=jax.ShapeDtypeStruct((NUM_TILES, N_PAD), jnp.float32),
    mesh=_mesh,
    compiler_params=pltpu.CompilerParams(needs_layout_passes=False),
    scratch_types=[
        pltpu.VMEM((4, RB, LANES), jnp.int32),
        pltpu.VMEM((N_PAD,), jnp.float32),
        pltpu.SemaphoreType.DMA,
    ],
)(_deg_body)


def _gs_body(src_hbm, dst_hbm, g_hbm, zeros_hbm, out_hbm,
             sbuf, dbuf, vbuf, gcopy, tsh, sem_i, sem_s):
    cid = lax.axis_index("c")
    sid = lax.axis_index("s")
    wid = cid * 16 + sid
    is_last = wid == NUM_TILES - 1
    nblk_w = jnp.where(is_last, NBLK_LAST, NBLK)
    nbase = sid * NSLICE
    rbase = wid * ROWS_PER_TILE

    def _fetch(b, slot):
        row0 = rbase + b * RB
        pltpu.async_copy(src_hbm.at[pl.ds(row0, RB)], sbuf.at[slot], sem_i)
        pltpu.async_copy(dst_hbm.at[pl.ds(row0, RB)], dbuf.at[slot], sem_i)

    def _wait_fetch(b, slot):
        row0 = rbase + b * RB
        pltpu.make_async_copy(src_hbm.at[pl.ds(row0, RB)], sbuf.at[slot],
                              sem_i).wait()
        pltpu.make_async_copy(dst_hbm.at[pl.ds(row0, RB)], dbuf.at[slot],
                              sem_i).wait()

    def _gather(slot, q):
        for r in range(RB):
            for j in range(LANES // 16):
                s16 = sbuf[slot, r, pl.ds(j * 16, 16)]
                vbuf[q, r, pl.ds(j * 16, 16)] = plsc.load_gather(gcopy, [s16])

    def _issue_scatter(slot, q):
        for r in range(RB):
            pltpu.async_copy(vbuf.at[q, r], tsh.at[dbuf.at[slot, r]], sem_s,
                             add=True)

    def _wait_scatter(slot, q):
        for r in range(RB):
            pltpu.make_async_copy(vbuf.at[q, r], tsh.at[dbuf.at[slot, r]],
                                  sem_s).wait()

    for b in range(3):
        _fetch(b, b)
    pltpu.async_copy(zeros_hbm.at[pl.ds(nbase, NSLICE)],
                     tsh.at[pl.ds(nbase, NSLICE)], sem_s)
    pltpu.async_copy(g_hbm, gcopy, sem_s)
    pltpu.make_async_copy(zeros_hbm.at[pl.ds(nbase, NSLICE)],
                          tsh.at[pl.ds(nbase, NSLICE)], sem_s).wait()
    pltpu.make_async_copy(g_hbm, gcopy, sem_s).wait()
    plsc.subcore_barrier()

    def quad(ii, carry):
        b0 = ii * 4
        for k in range(4):
            slot = k
            q = k % 2
            _wait_fetch(b0 + k, slot)
            if k == 0:
                @pl.when(ii > 0)
                def _():
                    _wait_scatter(3, 1)
            else:
                _wait_scatter(k - 1, (k - 1) % 2)

            @pl.when(b0 + k + 3 < nblk_w)
            def _():
                _fetch(b0 + k + 3, (k + 3) % 4)

            _gather(slot, q)
            _issue_scatter(slot, q)
        return carry

    nq_w = jnp.where(is_last, NBLK_LAST // 4, NQUAD)
    lax.fori_loop(0, nq_w, quad, 0)

    @pl.when(is_last)
    def _():
        _wait_fetch(NBLK_LAST - 1, (NBLK_LAST - 1) % 4)
        _wait_scatter(3, 1)
        _gather((NBLK_LAST - 1) % 4, (NBLK_LAST - 1) % 2)
        _issue_scatter((NBLK_LAST - 1) % 4, (NBLK_LAST - 1) % 2)
        _wait_scatter((NBLK_LAST - 1) % 4, (NBLK_LAST - 1) % 2)

    @pl.when(jnp.logical_not(is_last))
    def _():
        _wait_scatter((NBLK - 1) % 4, (NBLK - 1) % 2)

    plsc.subcore_barrier()
    pltpu.sync_copy(tsh.at[pl.ds(nbase, NSLICE)],
                    out_hbm.at[cid, pl.ds(nbase, NSLICE)])


_gs_kernel = functools.partial(
    pl.kernel,
    out_type=jax.ShapeDtypeStruct((2, N_PAD), jnp.float32),
    mesh=_mesh,
    compiler_params=pltpu.CompilerParams(needs_layout_passes=False),
    scratch_types=[
        pltpu.VMEM((4, RB, LANES), jnp.int32),
        pltpu.VMEM((4, RB, LANES), jnp.int32),
        pltpu.VMEM((2, RB, LANES), jnp.float32),
        pltpu.VMEM((N_PAD,), jnp.float32),
        pltpu.VMEM_SHARED((N_PAD,), jnp.float32),
        pltpu.SemaphoreType.DMA,
        pltpu.SemaphoreType.DMA,
    ],
)(_gs_body)


_R2 = N_PAD // LANES


def _prep_body(deg32_ref, x_ref, dinv_ref, g_ref):
    deg = jnp.sum(deg32_ref[...], axis=0) + 1.0
    dinv = lax.rsqrt(deg)
    dinv_ref[...] = dinv
    g_ref[...] = dinv * x_ref[...]


_prep_call = pl.pallas_call(
    _prep_body,
    out_shape=(jax.ShapeDtypeStruct((_R2, LANES), jnp.float32),
               jax.ShapeDtypeStruct((_R2, LANES), jnp.float32)),
)


def _mid_body(t1_ref, dinv_ref, g_ref, w1_ref, b1_ref, w2_ref,
              h2_ref, g2_ref):
    dinv = dinv_ref[...]
    s1 = dinv * (t1_ref[0] + t1_ref[1] + g_ref[...])
    acc = jnp.zeros_like(s1)
    for k in range(16):
        acc = acc + jnp.maximum(s1 * w1_ref[0, k] + b1_ref[k], 0.0) * w2_ref[k, 0]
    h2_ref[...] = acc
    g2_ref[...] = dinv * acc


_mid_call = pl.pallas_call(
    _mid_body,
    in_specs=[
        pl.BlockSpec(memory_space=pltpu.VMEM),
        pl.BlockSpec(memory_space=pltpu.VMEM),
        pl.BlockSpec(memory_space=pltpu.VMEM),
        pl.BlockSpec(memory_space=pltpu.SMEM),
        pl.BlockSpec(memory_space=pltpu.SMEM),
        pl.BlockSpec(memory_space=pltpu.SMEM),
    ],
    out_shape=(jax.ShapeDtypeStruct((_R2, LANES), jnp.float32),
               jax.ShapeDtypeStruct((_R2, LANES), jnp.float32)),
)


def _fin_body(t2_ref, dinv_ref, g2_ref, b2_ref, out_ref):
    out_ref[...] = dinv_ref[...] * (t2_ref[0] + t2_ref[1] + g2_ref[...]) + b2_ref[0]


_fin_call = pl.pallas_call(
    _fin_body,
    in_specs=[
        pl.BlockSpec(memory_space=pltpu.VMEM),
        pl.BlockSpec(memory_space=pltpu.VMEM),
        pl.BlockSpec(memory_space=pltpu.VMEM),
        pl.BlockSpec(memory_space=pltpu.SMEM),
    ],
    out_shape=jax.ShapeDtypeStruct((_R2, LANES), jnp.float32),
)


def kernel(x, edge_index, W1, b1, W2, b2):
    ei = edge_index.astype(jnp.int32)
    src = ei[0].reshape(ROWS_TOTAL, LANES)
    dst = ei[1].reshape(ROWS_TOTAL, LANES)
    xp = jnp.pad(x[:, 0], (0, N_PAD - N_NODES))
    zeros = jnp.zeros((N_PAD,), jnp.float32)

    deg32 = _deg_kernel(dst, zeros)
    dinv, g = _prep_call(deg32.reshape(NUM_TILES, _R2, LANES),
                         xp.reshape(_R2, LANES))
    t1 = _gs_kernel(src, dst, g.reshape(N_PAD), zeros)
    h2, g2 = _mid_call(t1.reshape(2, _R2, LANES), dinv, g, W1, b1, W2)
    t2 = _gs_kernel(src, dst, g2.reshape(N_PAD), zeros)
    out = _fin_call(t2.reshape(2, _R2, LANES), dinv, g2, b2)
    return out.reshape(N_PAD)[:N_NODES].reshape(N_NODES, 1)
